# baseline (speedup 1.0000x reference)
import dataclasses
import functools

import jax
import jax.numpy as jnp
from jax import lax
from jax.experimental import pallas as pl
from jax.experimental.pallas import tpu as pltpu
from jax.experimental.pallas import tpu_sc as plsc

_N = 320000
_D = 128
_S = 512
_NC = 2
_NS = 16
_NW = _NC * _NS
_LANES = 16

_SC_ROWS = 256000
_BT = 3200
_CH = 160
_RPT = _SC_ROWS // _NW
_NCH = _RPT // _CH

_TC_B = 1280
_TC_OFF = _SC_ROWS // _TC_B
_TC_NB = (_N - _SC_ROWS) // _TC_B


def _z_body(x_ref, w1_ref, b1_ref, z_ref):
    z_ref[...] = jnp.maximum(
        jnp.dot(x_ref[...], w1_ref[...], preferred_element_type=jnp.float32)
        + b1_ref[...], 0.0)


def _tc_z(x, w1, b1):
    return pl.pallas_call(
        _z_body,
        grid=(_SC_ROWS // _BT,),
        in_specs=[
            pl.BlockSpec((_BT, _D), lambda i: (i, 0)),
            pl.BlockSpec((_D, _D), lambda i: (0, 0)),
            pl.BlockSpec((1, _D), lambda i: (0, 0)),
        ],
        out_specs=pl.BlockSpec((_BT, _D), lambda i: (i, 0)),
        out_shape=jax.ShapeDtypeStruct((_SC_ROWS, _D), jnp.float32),
    )(x, w1, b1)


_vmesh = plsc.VectorSubcoreMesh(core_axis_name="c", subcore_axis_name="s")

_sc_params = pltpu.CompilerParams()
if "needs_layout_passes" in pltpu.CompilerParams.__dataclass_fields__:
    _sc_params = dataclasses.replace(_sc_params, needs_layout_passes=False)


@functools.partial(
    pl.kernel,
    out_type=[
        jax.ShapeDtypeStruct((_NC, _S, _D), jnp.float32),
        jax.ShapeDtypeStruct((_NW, _S, _LANES), jnp.float32),
    ],
    mesh=_vmesh,
    compiler_params=_sc_params,
    scratch_types=[
        pltpu.VMEM((_CH, _D), jnp.float32),
        pltpu.VMEM((_CH, _D), jnp.float32),
        pltpu.VMEM((_CH,), jnp.int32),
        pltpu.VMEM((_CH,), jnp.int32),
        pltpu.VMEM((_S, _LANES), jnp.float32),
        pltpu.VMEM_SHARED((_S, _D), jnp.float32),
        pltpu.SemaphoreType.DMA,
        pltpu.SemaphoreType.DMA,
        pltpu.SemaphoreType.DMA,
        pltpu.SemaphoreType.DMA,
    ],
)
def _sc_reduce(z_hbm, ids_hbm, psum_hbm, pcnt_hbm, buf0, buf1, ids0, ids1,
               cntl, sacc, zs0, zs1, is0, is1):
    cid = lax.axis_index("c")
    sid = lax.axis_index("s")
    wid = sid * _NC + cid
    base = wid * _RPT
    iota16 = lax.broadcasted_iota(jnp.int32, (_LANES,), 0)
    ones16 = jnp.ones((_LANES,), jnp.float32)
    bufs, idss, zsems, isems = (buf0, buf1), (ids0, ids1), (zs0, zs1), (is0, is1)

    @pl.loop(0, _S)
    def _zc(r):
        cntl.at[r, pl.ds(0, _LANES)][...] = jnp.zeros((_LANES,), jnp.float32)

    @pl.loop(0, _CH)
    def _fill(r):
        for c in range(_D // _LANES):
            buf0.at[r, pl.ds(c * _LANES, _LANES)][...] = (
                jnp.zeros((_LANES,), jnp.float32))

    @pl.when(sid == 0)
    def _init_shared():
        for q in range(_S // _CH + (1 if _S % _CH else 0)):
            n = min(_CH, _S - q * _CH)
            pltpu.sync_copy(buf0.at[pl.ds(0, n)], sacc.at[pl.ds(q * _CH, n)])

    plsc.subcore_barrier()

    def _start_load(k, b):
        off = base + k * _CH
        pltpu.async_copy(z_hbm.at[pl.ds(off, _CH)], bufs[b], zsems[b])
        pltpu.async_copy(ids_hbm.at[pl.ds(off, _CH)], idss[b], isems[b])

    def _wait_load(k, b):
        off = base + k * _CH
        pltpu.make_async_copy(z_hbm.at[pl.ds(off, _CH)], bufs[b],
                              zsems[b]).wait()
        pltpu.make_async_copy(ids_hbm.at[pl.ds(off, _CH)], idss[b],
                              isems[b]).wait()

    def _consume(b):
        pltpu.sync_copy(bufs[b], sacc.at[idss[b]], add=True)

        @pl.loop(0, _CH // _LANES)
        def _cnt(g):
            ids16 = idss[b].at[pl.ds(g * _LANES, _LANES)][...]
            plsc.addupdate_scatter(cntl, [ids16, iota16], ones16)

    _start_load(0, 0)

    @pl.loop(0, _NCH, step=2)
    def _chunk(k):
        _wait_load(k, 0)
        _start_load(k + 1, 1)
        _consume(0)
        _wait_load(k + 1, 1)

        @pl.when(k + 2 < _NCH)
        def _pref():
            _start_load(k + 2, 0)
        _consume(1)

    plsc.subcore_barrier()
    pltpu.sync_copy(cntl, pcnt_hbm.at[wid])

    @pl.when(sid == 0)
    def _writeback():
        pltpu.sync_copy(sacc, psum_hbm.at[cid])


def _oh_body(x_ref, ids_ref, w1_ref, b1_ref, acc_out, cnt_out,
             acc_ref, cnt_ref):
    i = pl.program_id(0)
    nb = pl.num_programs(0)

    @pl.when(i == 0)
    def _init():
        acc_ref[...] = jnp.zeros_like(acc_ref)
        cnt_ref[...] = jnp.zeros_like(cnt_ref)

    y = jnp.maximum(
        jnp.dot(x_ref[...], w1_ref[...], preferred_element_type=jnp.float32)
        + b1_ref[...], 0.0)

    ids = ids_ref[0, 0, :].reshape(1, _TC_B)
    eq = ids == lax.broadcasted_iota(jnp.int32, (_S, _TC_B), 0)
    oht = eq.astype(jnp.bfloat16)

    acc_ref[...] += jnp.dot(oht, y.astype(jnp.bfloat16),
                            preferred_element_type=jnp.float32)
    cnt_ref[...] += jnp.sum(eq.astype(jnp.float32), axis=1, keepdims=True)

    @pl.when(i == nb - 1)
    def _finish():
        acc_out[...] = acc_ref[...]
        cnt_out[...] = cnt_ref[...]


def _tc_onehot(x, ids3, w1, b1):
    return pl.pallas_call(
        _oh_body,
        grid=(_TC_NB,),
        in_specs=[
            pl.BlockSpec((_TC_B, _D), lambda i: (i + _TC_OFF, 0)),
            pl.BlockSpec((1, 1, _TC_B), lambda i: (i + _TC_OFF, 0, 0)),
            pl.BlockSpec((_D, _D), lambda i: (0, 0)),
            pl.BlockSpec((1, _D), lambda i: (0, 0)),
        ],
        out_specs=[
            pl.BlockSpec((_S, _D), lambda i: (0, 0)),
            pl.BlockSpec((_S, 1), lambda i: (0, 0)),
        ],
        out_shape=[
            jax.ShapeDtypeStruct((_S, _D), jnp.float32),
            jax.ShapeDtypeStruct((_S, 1), jnp.float32),
        ],
        scratch_shapes=[
            pltpu.VMEM((_S, _D), jnp.float32),
            pltpu.VMEM((_S, 1), jnp.float32),
        ],
    )(x, ids3, w1, b1)


def _fin_body(ps_ref, pc_ref, acc_tc_ref, cnt_tc_ref, w2_ref, b2_ref,
              out_ref):
    acc = jnp.sum(ps_ref[...], axis=0) + acc_tc_ref[...]
    cnt = jnp.sum(pc_ref[...], axis=(0, 2))[:, None] + cnt_tc_ref[...]
    pooled = acc / jnp.maximum(cnt, 1.0)
    out_ref[...] = (
        jnp.dot(pooled, w2_ref[...], preferred_element_type=jnp.float32)
        + b2_ref[...])


def _tc_fin(ps, pc, acc_tc, cnt_tc, w2, b2):
    return pl.pallas_call(
        _fin_body,
        in_specs=[
            pl.BlockSpec((_NC, _S, _D), lambda: (0, 0, 0)),
            pl.BlockSpec((_NW, _S, _LANES), lambda: (0, 0, 0)),
            pl.BlockSpec((_S, _D), lambda: (0, 0)),
            pl.BlockSpec((_S, 1), lambda: (0, 0)),
            pl.BlockSpec((_D, _D), lambda: (0, 0)),
            pl.BlockSpec((1, _D), lambda: (0, 0)),
        ],
        out_specs=pl.BlockSpec((_S, _D), lambda: (0, 0)),
        out_shape=jax.ShapeDtypeStruct((_S, _D), jnp.float32),
    )(ps, pc, acc_tc, cnt_tc, w2, b2)


def kernel(input, batch, emb_weight, emb_bias, mlp_weight, mlp_bias):
    ids = batch.astype(jnp.int32)
    b1 = emb_bias.reshape(1, _D)
    z = _tc_z(input, emb_weight, b1)
    ids3 = ids.reshape(_N // _TC_B, 1, _TC_B)
    ps, pc = _sc_reduce(z, ids)
    acc_tc, cnt_tc = _tc_onehot(input, ids3, emb_weight, b1)
    return _tc_fin(ps, pc, acc_tc, cnt_tc, mlp_weight,
                   mlp_bias.reshape(1, _D))


# speedup vs baseline: 1.0826x; 1.0826x over previous
import dataclasses
import functools

import jax
import jax.numpy as jnp
from jax import lax
from jax.experimental import pallas as pl
from jax.experimental.pallas import tpu as pltpu
from jax.experimental.pallas import tpu_sc as plsc

_N = 320000
_D = 128
_S = 512
_NC = 2
_NS = 16
_NW = _NC * _NS
_LANES = 16

_SC_ROWS = 204800
_BT = 3200
_CH = 160
_RPT = _SC_ROWS // _NW
_NCH = _RPT // _CH

_TC_B = 1280
_TC_OFF = _SC_ROWS // _TC_B
_TC_NB = (_N - _SC_ROWS) // _TC_B


def _z_body(x_ref, w1_ref, b1_ref, z_ref):
    z_ref[...] = jnp.maximum(
        jnp.dot(x_ref[...], w1_ref[...], preferred_element_type=jnp.float32)
        + b1_ref[...], 0.0)


def _tc_z(x, w1, b1):
    return pl.pallas_call(
        _z_body,
        grid=(_SC_ROWS // _BT,),
        in_specs=[
            pl.BlockSpec((_BT, _D), lambda i: (i, 0)),
            pl.BlockSpec((_D, _D), lambda i: (0, 0)),
            pl.BlockSpec((1, _D), lambda i: (0, 0)),
        ],
        out_specs=pl.BlockSpec((_BT, _D), lambda i: (i, 0)),
        out_shape=jax.ShapeDtypeStruct((_SC_ROWS, _D), jnp.float32),
    )(x, w1, b1)


_vmesh = plsc.VectorSubcoreMesh(core_axis_name="c", subcore_axis_name="s")

_sc_params = pltpu.CompilerParams()
if "needs_layout_passes" in pltpu.CompilerParams.__dataclass_fields__:
    _sc_params = dataclasses.replace(_sc_params, needs_layout_passes=False)


@functools.partial(
    pl.kernel,
    out_type=[
        jax.ShapeDtypeStruct((_NC, _S, _D), jnp.float32),
        jax.ShapeDtypeStruct((_NW, _S, _LANES), jnp.float32),
    ],
    mesh=_vmesh,
    compiler_params=_sc_params,
    scratch_types=[
        pltpu.VMEM((_CH, _D), jnp.float32),
        pltpu.VMEM((_CH, _D), jnp.float32),
        pltpu.VMEM((_CH,), jnp.int32),
        pltpu.VMEM((_CH,), jnp.int32),
        pltpu.VMEM((_S, _LANES), jnp.float32),
        pltpu.VMEM_SHARED((_S, _D), jnp.float32),
        pltpu.SemaphoreType.DMA,
        pltpu.SemaphoreType.DMA,
        pltpu.SemaphoreType.DMA,
        pltpu.SemaphoreType.DMA,
    ],
)
def _sc_reduce(z_hbm, ids_hbm, psum_hbm, pcnt_hbm, buf0, buf1, ids0, ids1,
               cntl, sacc, zs0, zs1, is0, is1):
    cid = lax.axis_index("c")
    sid = lax.axis_index("s")
    wid = sid * _NC + cid
    base = wid * _RPT
    iota16 = lax.broadcasted_iota(jnp.int32, (_LANES,), 0)
    ones16 = jnp.ones((_LANES,), jnp.float32)
    bufs, idss, zsems, isems = (buf0, buf1), (ids0, ids1), (zs0, zs1), (is0, is1)

    @pl.loop(0, _S)
    def _zc(r):
        cntl.at[r, pl.ds(0, _LANES)][...] = jnp.zeros((_LANES,), jnp.float32)

    @pl.loop(0, _CH)
    def _fill(r):
        for c in range(_D // _LANES):
            buf0.at[r, pl.ds(c * _LANES, _LANES)][...] = (
                jnp.zeros((_LANES,), jnp.float32))

    @pl.when(sid == 0)
    def _init_shared():
        for q in range(_S // _CH + (1 if _S % _CH else 0)):
            n = min(_CH, _S - q * _CH)
            pltpu.sync_copy(buf0.at[pl.ds(0, n)], sacc.at[pl.ds(q * _CH, n)])

    plsc.subcore_barrier()

    def _start_load(k, b):
        off = base + k * _CH
        pltpu.async_copy(z_hbm.at[pl.ds(off, _CH)], bufs[b], zsems[b])
        pltpu.async_copy(ids_hbm.at[pl.ds(off, _CH)], idss[b], isems[b])

    def _wait_load(k, b):
        off = base + k * _CH
        pltpu.make_async_copy(z_hbm.at[pl.ds(off, _CH)], bufs[b],
                              zsems[b]).wait()
        pltpu.make_async_copy(ids_hbm.at[pl.ds(off, _CH)], idss[b],
                              isems[b]).wait()

    def _consume(b):
        pltpu.sync_copy(bufs[b], sacc.at[idss[b]], add=True)

        @pl.loop(0, _CH // _LANES)
        def _cnt(g):
            ids16 = idss[b].at[pl.ds(g * _LANES, _LANES)][...]
            plsc.addupdate_scatter(cntl, [ids16, iota16], ones16)

    _start_load(0, 0)

    @pl.loop(0, _NCH, step=2)
    def _chunk(k):
        _wait_load(k, 0)
        _start_load(k + 1, 1)
        _consume(0)
        _wait_load(k + 1, 1)

        @pl.when(k + 2 < _NCH)
        def _pref():
            _start_load(k + 2, 0)
        _consume(1)

    plsc.subcore_barrier()
    pltpu.sync_copy(cntl, pcnt_hbm.at[wid])

    @pl.when(sid == 0)
    def _writeback():
        pltpu.sync_copy(sacc, psum_hbm.at[cid])


def _oh_body(x_ref, ids_ref, w1_ref, b1_ref, acc_out, cnt_out,
             acc_ref, cnt_ref):
    i = pl.program_id(0)
    nb = pl.num_programs(0)

    @pl.when(i == 0)
    def _init():
        acc_ref[...] = jnp.zeros_like(acc_ref)
        cnt_ref[...] = jnp.zeros_like(cnt_ref)

    y = jnp.maximum(
        jnp.dot(x_ref[...], w1_ref[...], preferred_element_type=jnp.float32)
        + b1_ref[...], 0.0)

    ids = ids_ref[0, 0, :].reshape(1, _TC_B)
    eq = ids == lax.broadcasted_iota(jnp.int32, (_S, _TC_B), 0)
    oht = eq.astype(jnp.bfloat16)

    acc_ref[...] += jnp.dot(oht, y.astype(jnp.bfloat16),
                            preferred_element_type=jnp.float32)
    cnt_ref[...] += jnp.sum(eq.astype(jnp.float32), axis=1, keepdims=True)

    @pl.when(i == nb - 1)
    def _finish():
        acc_out[...] = acc_ref[...]
        cnt_out[...] = cnt_ref[...]


def _tc_onehot(x, ids3, w1, b1):
    return pl.pallas_call(
        _oh_body,
        grid=(_TC_NB,),
        in_specs=[
            pl.BlockSpec((_TC_B, _D), lambda i: (i + _TC_OFF, 0)),
            pl.BlockSpec((1, 1, _TC_B), lambda i: (i + _TC_OFF, 0, 0)),
            pl.BlockSpec((_D, _D), lambda i: (0, 0)),
            pl.BlockSpec((1, _D), lambda i: (0, 0)),
        ],
        out_specs=[
            pl.BlockSpec((_S, _D), lambda i: (0, 0)),
            pl.BlockSpec((_S, 1), lambda i: (0, 0)),
        ],
        out_shape=[
            jax.ShapeDtypeStruct((_S, _D), jnp.float32),
            jax.ShapeDtypeStruct((_S, 1), jnp.float32),
        ],
        scratch_shapes=[
            pltpu.VMEM((_S, _D), jnp.float32),
            pltpu.VMEM((_S, 1), jnp.float32),
        ],
    )(x, ids3, w1, b1)


def _fin_body(ps_ref, pc_ref, acc_tc_ref, cnt_tc_ref, w2_ref, b2_ref,
              out_ref):
    acc = jnp.sum(ps_ref[...], axis=0) + acc_tc_ref[...]
    cnt = jnp.sum(pc_ref[...], axis=(0, 2))[:, None] + cnt_tc_ref[...]
    pooled = acc / jnp.maximum(cnt, 1.0)
    out_ref[...] = (
        jnp.dot(pooled, w2_ref[...], preferred_element_type=jnp.float32)
        + b2_ref[...])


def _tc_fin(ps, pc, acc_tc, cnt_tc, w2, b2):
    return pl.pallas_call(
        _fin_body,
        in_specs=[
            pl.BlockSpec((_NC, _S, _D), lambda: (0, 0, 0)),
            pl.BlockSpec((_NW, _S, _LANES), lambda: (0, 0, 0)),
            pl.BlockSpec((_S, _D), lambda: (0, 0)),
            pl.BlockSpec((_S, 1), lambda: (0, 0)),
            pl.BlockSpec((_D, _D), lambda: (0, 0)),
            pl.BlockSpec((1, _D), lambda: (0, 0)),
        ],
        out_specs=pl.BlockSpec((_S, _D), lambda: (0, 0)),
        out_shape=jax.ShapeDtypeStruct((_S, _D), jnp.float32),
    )(ps, pc, acc_tc, cnt_tc, w2, b2)


def kernel(input, batch, emb_weight, emb_bias, mlp_weight, mlp_bias):
    ids = batch.astype(jnp.int32)
    b1 = emb_bias.reshape(1, _D)
    z = _tc_z(input, emb_weight, b1)
    ids3 = ids.reshape(_N // _TC_B, 1, _TC_B)
    ps, pc = _sc_reduce(z, ids)
    acc_tc, cnt_tc = _tc_onehot(input, ids3, emb_weight, b1)
    return _tc_fin(ps, pc, acc_tc, cnt_tc, mlp_weight,
                   mlp_bias.reshape(1, _D))
